# baseline (speedup 1.0000x reference)
import functools

import jax
import jax.numpy as jnp
from jax import lax
from jax.experimental import pallas as pl
from jax.experimental.pallas import tpu as pltpu
from jax.experimental.pallas import tpu_sc as plsc

_NUM_CORES = 2
_NUM_SUBCORES = 16
_NUM_WORKERS = _NUM_CORES * _NUM_SUBCORES
_LANE = 128
_VREG = 16


@jax.jit
def _sc_embedding_lookup(ids_p, tab_wide):
    seq, batch = ids_p.shape
    d = tab_wide.shape[1] // 2
    cols = batch // _NUM_WORKERS
    n_q = cols // _LANE
    n_blocks = n_q * seq
    assert cols * _NUM_WORKERS == batch and n_q * _LANE == cols
    assert n_blocks % 2 == 0 and d % _VREG == 0

    mesh = plsc.VectorSubcoreMesh(core_axis_name="c", subcore_axis_name="s")

    @functools.partial(
        pl.kernel,
        out_type=jax.ShapeDtypeStruct((seq, d, batch), jnp.float32),
        mesh=mesh,
        scratch_types=[
            pltpu.VMEM((56, _LANE), jnp.int32),
            pltpu.VMEM((n_blocks, _LANE), jnp.int32),
            pltpu.VMEM((n_blocks, _LANE), jnp.int32),
            pltpu.VMEM((2, _LANE, 2 * d), jnp.float32),
            pltpu.VMEM((2, d, _LANE), jnp.float32),
            pltpu.SemaphoreType.DMA,
            pltpu.SemaphoreType.DMA,
            pltpu.SemaphoreType.DMA,
            pltpu.SemaphoreType.DMA,
        ],
        compiler_params=pltpu.CompilerParams(
            use_tc_tiling_on_sc=True, needs_layout_passes=False
        ),
    )
    def lookup(
        ids_hbm, tab_hbm, out_hbm, stage_v, idx_v, par_v, rows_v, trans_v,
        g0, g1, o0, o1,
    ):
        wid = lax.axis_index("s") * _NUM_CORES + lax.axis_index("c")
        col0 = wid * cols
        gsem = (g0, g1)
        osem = (o0, o1)

        for q in range(n_q):
            pltpu.sync_copy(
                ids_hbm.at[:, pl.ds(col0 + q * _LANE, _LANE)],
                stage_v.at[pl.ds(0, seq)],
            )

            @plsc.parallel_loop(0, seq, 1, unroll=2)
            def stage_body(s):
                for m in range(_LANE // _VREG):
                    v = stage_v[s, pl.ds(m * _VREG, _VREG)]
                    idx_v[q * seq + s, pl.ds(m * _VREG, _VREG)] = v >> 1
                    par_v[q * seq + s, pl.ds(m * _VREG, _VREG)] = (v & 1) * d

        def fire_gather(b, slot):
            pltpu.async_copy(tab_hbm.at[idx_v.at[b]], rows_v.at[slot], gsem[slot])

        def drain_gather(slot):
            pltpu.make_async_copy(
                tab_hbm.at[pl.ds(0, _LANE)], rows_v.at[slot], gsem[slot]
            ).wait()

        def transpose(b, slot):
            src = rows_v.at[slot]
            dst = trans_v.at[slot]
            half = [par_v[b, pl.ds(m * _VREG, _VREG)] for m in range(_LANE // _VREG)]
            rowc = [
                lax.iota(jnp.int32, _VREG) + (m * _VREG)
                for m in range(_LANE // _VREG)
            ]

            @plsc.parallel_loop(0, d, 1, unroll=16)
            def t_body(e):
                for m in range(_LANE // _VREG):
                    dst[e, pl.ds(m * _VREG, _VREG)] = plsc.load_gather(
                        src, [rowc[m], half[m] + e]
                    )

        def fire_out(b, slot):
            s = b % seq
            q = b // seq
            pltpu.async_copy(
                trans_v.at[slot],
                out_hbm.at[s, :, pl.ds(col0 + q * _LANE, _LANE)],
                osem[slot],
            )

        def drain_out(slot):
            pltpu.make_async_copy(
                trans_v.at[slot], out_hbm.at[0, :, pl.ds(0, _LANE)], osem[slot]
            ).wait()

        fire_gather(0, 0)
        fire_gather(1, 1)
        for b in (0, 1):
            drain_gather(b)
            transpose(b, b)
            fire_gather(2 + b, b)
            fire_out(b, b)

        def body(i, carry):
            b0 = 2 + 2 * i
            for sl in (0, 1):
                b = b0 + sl
                drain_gather(sl)
                drain_out(sl)
                transpose(b, sl)
                fire_gather(b + 2, sl)
                fire_out(b, sl)
            return carry

        lax.fori_loop(0, (n_blocks - 4) // 2, body, 0)

        for sl in (0, 1):
            b = n_blocks - 2 + sl
            drain_gather(sl)
            drain_out(sl)
            transpose(b, sl)
            fire_out(b, sl)
        drain_out(0)
        drain_out(1)

    return lookup(ids_p, tab_wide)


def kernel(token_ids, embedding_table):
    vocab, d = embedding_table.shape
    ids_p = jnp.transpose(token_ids).astype(jnp.int32)
    tab_wide = embedding_table.reshape(vocab // 2, 2 * d)
    out_p = _sc_embedding_lookup(ids_p, tab_wide)
    return jnp.transpose(out_p, (2, 0, 1))


# speedup vs baseline: 1.4629x; 1.4629x over previous
import functools

import jax
import jax.numpy as jnp
from jax import lax
from jax.experimental import pallas as pl
from jax.experimental.pallas import tpu as pltpu
from jax.experimental.pallas import tpu_sc as plsc

_NUM_CORES = 2
_NUM_SUBCORES = 16
_NUM_WORKERS = _NUM_CORES * _NUM_SUBCORES
_LANE = 128
_VREG = 16


@jax.jit
def _sc_embedding_lookup(ids_p, tab_wide):
    seq, batch = ids_p.shape
    d = tab_wide.shape[1] // 2
    cols = batch // _NUM_WORKERS
    n_q = cols // _LANE
    n_blocks = n_q * seq
    assert cols * _NUM_WORKERS == batch and n_q * _LANE == cols
    assert n_blocks % 2 == 0 and d % _VREG == 0

    mesh = plsc.VectorSubcoreMesh(core_axis_name="c", subcore_axis_name="s")

    @functools.partial(
        pl.kernel,
        out_type=jax.ShapeDtypeStruct((seq, d, batch), jnp.float32),
        mesh=mesh,
        scratch_types=[
            pltpu.VMEM((56, _LANE), jnp.int32),
            pltpu.VMEM((n_blocks, _LANE), jnp.int32),
            pltpu.VMEM((n_blocks, _LANE), jnp.int32),
            pltpu.VMEM((2, _LANE, 2 * d), jnp.float32),
            pltpu.VMEM((2, d, _LANE), jnp.float32),
            pltpu.SemaphoreType.DMA,
            pltpu.SemaphoreType.DMA,
            pltpu.SemaphoreType.DMA,
            pltpu.SemaphoreType.DMA,
        ],
        compiler_params=pltpu.CompilerParams(
            use_tc_tiling_on_sc=True, needs_layout_passes=False
        ),
    )
    def lookup(
        ids_hbm, tab_hbm, out_hbm, stage_v, idx_v, par_v, rows_v, trans_v,
        g0, g1, o0, o1,
    ):
        wid = lax.axis_index("s") * _NUM_CORES + lax.axis_index("c")
        col0 = wid * cols
        gsem = (g0, g1)
        osem = (o0, o1)

        for q in range(n_q):
            pltpu.sync_copy(
                ids_hbm.at[:, pl.ds(col0 + q * _LANE, _LANE)],
                stage_v.at[pl.ds(0, seq)],
            )

            @plsc.parallel_loop(0, seq, 1, unroll=2)
            def stage_body(s):
                for m in range(_LANE // _VREG):
                    v = stage_v[s, pl.ds(m * _VREG, _VREG)]
                    idx_v[q * seq + s, pl.ds(m * _VREG, _VREG)] = v >> 1
                    par_v[q * seq + s, pl.ds(m * _VREG, _VREG)] = (v & 1) * d

        def fire_gather(b, slot):
            pltpu.async_copy(tab_hbm.at[idx_v.at[b]], rows_v.at[slot], gsem[slot])

        def drain_gather(slot):
            pltpu.make_async_copy(
                tab_hbm.at[pl.ds(0, _LANE)], rows_v.at[slot], gsem[slot]
            ).wait()

        def transpose(b, slot):
            src = rows_v.at[slot]
            dst = trans_v.at[slot]
            half = [par_v[b, pl.ds(m * _VREG, _VREG)] for m in range(_LANE // _VREG)]
            rowc = [
                lax.iota(jnp.int32, _VREG) + (m * _VREG)
                for m in range(_LANE // _VREG)
            ]

            @plsc.parallel_loop(0, _VREG, 1, unroll=4)
            def t_body(p):
                diag = (lax.iota(jnp.int32, _VREG) + p) & (_VREG - 1)
                for t in range(d // _VREG):
                    erow = diag + (t * _VREG)
                    for m in range(_LANE // _VREG):
                        v = plsc.load_gather(src, [rowc[m], half[m] + erow])
                        plsc.store_scatter(dst, [erow, rowc[m]], v)

        def fire_out(b, slot):
            s = b % seq
            q = b // seq
            pltpu.async_copy(
                trans_v.at[slot],
                out_hbm.at[s, :, pl.ds(col0 + q * _LANE, _LANE)],
                osem[slot],
            )

        def drain_out(slot):
            pltpu.make_async_copy(
                trans_v.at[slot], out_hbm.at[0, :, pl.ds(0, _LANE)], osem[slot]
            ).wait()

        fire_gather(0, 0)
        fire_gather(1, 1)
        for b in (0, 1):
            drain_gather(b)
            transpose(b, b)
            fire_gather(2 + b, b)
            fire_out(b, b)

        def body(i, carry):
            b0 = 2 + 2 * i
            for sl in (0, 1):
                b = b0 + sl
                drain_gather(sl)
                drain_out(sl)
                transpose(b, sl)
                fire_gather(b + 2, sl)
                fire_out(b, sl)
            return carry

        lax.fori_loop(0, (n_blocks - 4) // 2, body, 0)

        for sl in (0, 1):
            b = n_blocks - 2 + sl
            drain_gather(sl)
            drain_out(sl)
            transpose(b, sl)
            fire_out(b, sl)
        drain_out(0)
        drain_out(1)

    return lookup(ids_p, tab_wide)


def kernel(token_ids, embedding_table):
    vocab, d = embedding_table.shape
    ids_p = jnp.transpose(token_ids).astype(jnp.int32)
    tab_wide = embedding_table.reshape(vocab // 2, 2 * d)
    out_p = _sc_embedding_lookup(ids_p, tab_wide)
    return jnp.transpose(out_p, (2, 0, 1))


# speedup vs baseline: 1.4683x; 1.0037x over previous
import functools

import jax
import jax.numpy as jnp
from jax import lax
from jax.experimental import pallas as pl
from jax.experimental.pallas import tpu as pltpu
from jax.experimental.pallas import tpu_sc as plsc

_NUM_CORES = 2
_NUM_SUBCORES = 16
_NUM_WORKERS = _NUM_CORES * _NUM_SUBCORES
_LANE = 128
_VREG = 16


@jax.jit
def _sc_embedding_lookup(ids_p, tab_wide):
    seq, batch = ids_p.shape
    d = tab_wide.shape[1] // 2
    cols = batch // _NUM_WORKERS
    n_q = cols // _LANE
    n_blocks = n_q * seq
    assert cols * _NUM_WORKERS == batch and n_q * _LANE == cols
    assert n_blocks % 2 == 0 and d % _VREG == 0

    mesh = plsc.VectorSubcoreMesh(core_axis_name="c", subcore_axis_name="s")

    @functools.partial(
        pl.kernel,
        out_type=jax.ShapeDtypeStruct((seq, d, batch), jnp.float32),
        mesh=mesh,
        scratch_types=[
            pltpu.VMEM((56, _LANE), jnp.int32),
            pltpu.VMEM((n_blocks, _LANE), jnp.int32),
            pltpu.VMEM((n_blocks, _LANE), jnp.int32),
            pltpu.VMEM((2, _LANE, 2 * d), jnp.float32),
            pltpu.VMEM((2, d, _LANE), jnp.float32),
            pltpu.SemaphoreType.DMA,
            pltpu.SemaphoreType.DMA,
            pltpu.SemaphoreType.DMA,
            pltpu.SemaphoreType.DMA,
        ],
        compiler_params=pltpu.CompilerParams(
            use_tc_tiling_on_sc=True, needs_layout_passes=False
        ),
    )
    def lookup(
        ids_hbm, tab_hbm, out_hbm, stage_v, idx_v, par_v, rows_v, trans_v,
        g0, g1, o0, o1,
    ):
        wid = lax.axis_index("s") * _NUM_CORES + lax.axis_index("c")
        col0 = wid * cols
        gsem = (g0, g1)
        osem = (o0, o1)

        for q in range(n_q):
            pltpu.sync_copy(
                ids_hbm.at[:, pl.ds(col0 + q * _LANE, _LANE)],
                stage_v.at[pl.ds(0, seq)],
            )

            @plsc.parallel_loop(0, seq, 1, unroll=4)
            def stage_body(s):
                for m in range(_LANE // _VREG):
                    v = stage_v[s, pl.ds(m * _VREG, _VREG)]
                    idx_v[q * seq + s, pl.ds(m * _VREG, _VREG)] = v >> 1
                    par_v[q * seq + s, pl.ds(m * _VREG, _VREG)] = (v & 1) * d

        def fire_gather(b, slot):
            pltpu.async_copy(tab_hbm.at[idx_v.at[b]], rows_v.at[slot], gsem[slot])

        def drain_gather(slot):
            pltpu.make_async_copy(
                tab_hbm.at[pl.ds(0, _LANE)], rows_v.at[slot], gsem[slot]
            ).wait()

        def transpose(b, slot):
            src = rows_v.at[slot]
            dst = trans_v.at[slot]
            half = [par_v[b, pl.ds(m * _VREG, _VREG)] for m in range(_LANE // _VREG)]
            rowc = [
                lax.iota(jnp.int32, _VREG) + (m * _VREG)
                for m in range(_LANE // _VREG)
            ]

            @plsc.parallel_loop(0, _VREG, 1, unroll=8)
            def t_body(p):
                diag = (lax.iota(jnp.int32, _VREG) + p) & (_VREG - 1)
                for t in range(d // _VREG):
                    erow = diag + (t * _VREG)
                    for m in range(_LANE // _VREG):
                        v = plsc.load_gather(src, [rowc[m], half[m] + erow])
                        plsc.store_scatter(dst, [erow, rowc[m]], v)

        def fire_out(b, slot):
            s = b % seq
            q = b // seq
            pltpu.async_copy(
                trans_v.at[slot],
                out_hbm.at[s, :, pl.ds(col0 + q * _LANE, _LANE)],
                osem[slot],
            )

        def drain_out(slot):
            pltpu.make_async_copy(
                trans_v.at[slot], out_hbm.at[0, :, pl.ds(0, _LANE)], osem[slot]
            ).wait()

        fire_gather(0, 0)
        fire_gather(1, 1)
        for b in (0, 1):
            drain_gather(b)
            transpose(b, b)
            fire_gather(2 + b, b)
            fire_out(b, b)

        def body(i, carry):
            b0 = 2 + 2 * i
            for sl in (0, 1):
                b = b0 + sl
                drain_gather(sl)
                drain_out(sl)
                transpose(b, sl)
                fire_gather(b + 2, sl)
                fire_out(b, sl)
            return carry

        lax.fori_loop(0, (n_blocks - 4) // 2, body, 0)

        for sl in (0, 1):
            b = n_blocks - 2 + sl
            drain_gather(sl)
            drain_out(sl)
            transpose(b, sl)
            fire_out(b, sl)
        drain_out(0)
        drain_out(1)

    return lookup(ids_p, tab_wide)


def kernel(token_ids, embedding_table):
    vocab, d = embedding_table.shape
    ids_p = jnp.transpose(token_ids).astype(jnp.int32)
    tab_wide = embedding_table.reshape(vocab // 2, 2 * d)
    out_p = _sc_embedding_lookup(ids_p, tab_wide)
    return jnp.transpose(out_p, (2, 0, 1))


# speedup vs baseline: 1.4876x; 1.0132x over previous
import functools

import jax
import jax.numpy as jnp
from jax import lax
from jax.experimental import pallas as pl
from jax.experimental.pallas import tpu as pltpu
from jax.experimental.pallas import tpu_sc as plsc

_NUM_CORES = 2
_NUM_SUBCORES = 16
_NUM_WORKERS = _NUM_CORES * _NUM_SUBCORES
_LANE = 128
_VREG = 16


@jax.jit
def _sc_embedding_lookup(ids_p, tab_wide):
    seq, batch = ids_p.shape
    d = tab_wide.shape[1] // 2
    cols = batch // _NUM_WORKERS
    n_q = cols // _LANE
    n_blocks = n_q * seq
    assert cols * _NUM_WORKERS == batch and n_q * _LANE == cols
    assert n_blocks % 2 == 0 and d % _VREG == 0

    mesh = plsc.VectorSubcoreMesh(core_axis_name="c", subcore_axis_name="s")

    @functools.partial(
        pl.kernel,
        out_type=jax.ShapeDtypeStruct((seq, d, batch), jnp.float32),
        mesh=mesh,
        scratch_types=[
            pltpu.VMEM((56, _LANE), jnp.int32),
            pltpu.VMEM((n_blocks, _LANE), jnp.int32),
            pltpu.VMEM((n_blocks, _LANE), jnp.int32),
            pltpu.VMEM((3, _LANE, 2 * d), jnp.float32),
            pltpu.VMEM((2, d, _LANE), jnp.float32),
            pltpu.SemaphoreType.DMA,
            pltpu.SemaphoreType.DMA,
            pltpu.SemaphoreType.DMA,
            pltpu.SemaphoreType.DMA,
            pltpu.SemaphoreType.DMA,
        ],
        compiler_params=pltpu.CompilerParams(
            use_tc_tiling_on_sc=True, needs_layout_passes=False
        ),
    )
    def lookup(
        ids_hbm, tab_hbm, out_hbm, stage_v, idx_v, par_v, rows_v, trans_v,
        g0, g1, g2, o0, o1,
    ):
        wid = lax.axis_index("s") * _NUM_CORES + lax.axis_index("c")
        col0 = wid * cols
        gsem = (g0, g1, g2)
        osem = (o0, o1)

        for q in range(n_q):
            pltpu.sync_copy(
                ids_hbm.at[:, pl.ds(col0 + q * _LANE, _LANE)],
                stage_v.at[pl.ds(0, seq)],
            )

            @plsc.parallel_loop(0, seq, 1, unroll=4)
            def stage_body(s):
                for m in range(_LANE // _VREG):
                    v = stage_v[s, pl.ds(m * _VREG, _VREG)]
                    idx_v[q * seq + s, pl.ds(m * _VREG, _VREG)] = v >> 1
                    par_v[q * seq + s, pl.ds(m * _VREG, _VREG)] = (v & 1) * d

        def fire_gather(b, slot):
            pltpu.async_copy(tab_hbm.at[idx_v.at[b]], rows_v.at[slot], gsem[slot])

        def drain_gather(slot):
            pltpu.make_async_copy(
                tab_hbm.at[pl.ds(0, _LANE)], rows_v.at[slot], gsem[slot]
            ).wait()

        def transpose(b, rslot, tslot):
            src = rows_v.at[rslot]
            dst = trans_v.at[tslot]
            half = [par_v[b, pl.ds(m * _VREG, _VREG)] for m in range(_LANE // _VREG)]
            rowc = [
                lax.iota(jnp.int32, _VREG) + (m * _VREG)
                for m in range(_LANE // _VREG)
            ]

            @plsc.parallel_loop(0, _VREG, 1, unroll=8)
            def t_body(p):
                diag = (lax.iota(jnp.int32, _VREG) + p) & (_VREG - 1)
                for t in range(d // _VREG):
                    erow = diag + (t * _VREG)
                    for m in range(_LANE // _VREG):
                        v = plsc.load_gather(src, [rowc[m], half[m] + erow])
                        plsc.store_scatter(dst, [erow, rowc[m]], v)

        def fire_out(b, slot):
            s = b % seq
            q = b // seq
            pltpu.async_copy(
                trans_v.at[slot],
                out_hbm.at[s, :, pl.ds(col0 + q * _LANE, _LANE)],
                osem[slot],
            )

        def drain_out(slot):
            pltpu.make_async_copy(
                trans_v.at[slot], out_hbm.at[0, :, pl.ds(0, _LANE)], osem[slot]
            ).wait()

        fire_gather(0, 0)
        fire_gather(1, 1)
        fire_gather(2, 2)
        for b in (0, 1):
            drain_gather(b)
            transpose(b, b, b)
            fire_gather(3 + b, b)
            fire_out(b, b)

        def body(i, carry):
            b0 = 2 + 6 * i
            for j in range(6):
                b = b0 + j
                rs = (2 + j) % 3
                ts = j % 2
                drain_gather(rs)
                drain_out(ts)
                transpose(b, rs, ts)

                @pl.when(b + 3 < n_blocks)
                def _():
                    fire_gather(b + 3, rs)

                fire_out(b, ts)
            return carry

        lax.fori_loop(0, (n_blocks - 2) // 6, body, 0)

        drain_out(0)
        drain_out(1)

    return lookup(ids_p, tab_wide)


def kernel(token_ids, embedding_table):
    vocab, d = embedding_table.shape
    ids_p = jnp.transpose(token_ids).astype(jnp.int32)
    tab_wide = embedding_table.reshape(vocab // 2, 2 * d)
    out_p = _sc_embedding_lookup(ids_p, tab_wide)
    return jnp.transpose(out_p, (2, 0, 1))
